# baseline (speedup 1.0000x reference)
import functools

import jax
import jax.numpy as jnp
import numpy as np
from jax.experimental import pallas as pl
from jax.experimental.pallas import tpu as pltpu

_K_SEL = 5
_EPS = float(np.finfo(np.float32).tiny)
_NB = 4096
_CB = 25600


def _wexp_kernel(q_ref, n_ref, w_ref, s1_ref, *, n_blocks, k_valid):
    j = pl.program_id(0)
    q = q_ref[...]
    n = n_ref[...]

    q2 = jnp.sum(q * q, axis=1, keepdims=True)
    n2 = jax.lax.dot_general(
        jnp.ones((8, n.shape[1]), jnp.float32), n * n,
        (((1,), (1,)), ((), ())),
        preferred_element_type=jnp.float32)[0:1, :]
    qn = jax.lax.dot_general(
        q, n, (((1,), (1,)), ((), ())),
        preferred_element_type=jnp.float32)
    d2 = jnp.maximum(q2 + n2 - 2.0 * qn, 0.0)
    wb = jnp.exp(-jnp.sqrt(d2)).astype(jnp.bfloat16)
    w_ref[...] = wb
    w = wb.astype(jnp.float32)

    @pl.when(j == 0)
    def _():
        s1_ref[...] = jnp.zeros_like(s1_ref)

    last = n_blocks - 1
    valid = k_valid - last * _NB

    if valid < _NB:
        @pl.when(j < last)
        def _():
            s1_ref[:, 0:1] = s1_ref[:, 0:1] + jnp.sum(
                w, axis=1, keepdims=True)

        @pl.when(j == last)
        def _():
            w_ref[:, valid:] = jnp.zeros_like(w_ref[:, valid:])
            s1_ref[:, 0:1] = s1_ref[:, 0:1] + jnp.sum(
                w_ref[...].astype(jnp.float32), axis=1, keepdims=True)
    else:
        s1_ref[:, 0:1] = s1_ref[:, 0:1] + jnp.sum(w, axis=1, keepdims=True)


def _sweep_kernel(w0_ref, s1_ref, o_ref, w_ref, *, kp, cb, qb):
    s = s1_ref[:, 0:1]
    n_chunks = kp // cb

    for it in range(_K_SEL):
        rinv = 1.0 / s
        if it == 0:
            def body0(c, acc, _rinv=rinv):
                sl = pl.ds(c * cb, cb)
                wv = w0_ref[:, sl].astype(jnp.float32)
                oh = wv * _rinv
                o_ref[:, sl] = oh
                wn = wv * jnp.maximum(1.0 - oh, _EPS)
                w_ref[:, sl] = wn
                return acc + jnp.sum(wn, axis=1, keepdims=True)

            s = jax.lax.fori_loop(
                0, n_chunks, body0, jnp.zeros((qb, 1), jnp.float32))
        elif it < _K_SEL - 1:
            def body(c, acc, _rinv=rinv):
                sl = pl.ds(c * cb, cb)
                wv = w_ref[:, sl]
                oh = wv * _rinv
                o_ref[:, sl] = o_ref[:, sl] + oh
                wn = wv * jnp.maximum(1.0 - oh, _EPS)
                w_ref[:, sl] = wn
                return acc + jnp.sum(wn, axis=1, keepdims=True)

            s = jax.lax.fori_loop(
                0, n_chunks, body, jnp.zeros((qb, 1), jnp.float32))
        else:
            def body_last(c, acc, _rinv=rinv):
                sl = pl.ds(c * cb, cb)
                o_ref[:, sl] = o_ref[:, sl] + w_ref[:, sl] * _rinv
                return acc

            jax.lax.fori_loop(0, n_chunks, body_last, 0)


def kernel(query, neighbors):
    q_n, d = query.shape
    k_n, _ = neighbors.shape

    n_blocks = -(-k_n // _NB)
    kp = n_blocks * _NB
    cb = _CB if kp % _CB == 0 else _NB
    qb = 16
    while q_n % qb:
        qb //= 2

    w0, s1 = pl.pallas_call(
        functools.partial(_wexp_kernel, n_blocks=n_blocks, k_valid=k_n),
        grid=(n_blocks,),
        in_specs=[
            pl.BlockSpec((q_n, d), lambda j: (0, 0)),
            pl.BlockSpec((_NB, d), lambda j: (j, 0)),
        ],
        out_specs=[
            pl.BlockSpec((q_n, _NB), lambda j: (0, j)),
            pl.BlockSpec((q_n, 128), lambda j: (0, 0)),
        ],
        out_shape=[
            jax.ShapeDtypeStruct((q_n, kp), jnp.bfloat16),
            jax.ShapeDtypeStruct((q_n, 128), jnp.float32),
        ],
        compiler_params=pltpu.CompilerParams(
            dimension_semantics=("arbitrary",),
        ),
    )(query, neighbors)

    out = pl.pallas_call(
        functools.partial(_sweep_kernel, kp=kp, cb=cb, qb=qb),
        grid=(q_n // qb,),
        in_specs=[
            pl.BlockSpec((qb, kp), lambda i: (i, 0)),
            pl.BlockSpec((qb, 128), lambda i: (i, 0)),
        ],
        out_specs=pl.BlockSpec((qb, kp), lambda i: (i, 0)),
        out_shape=jax.ShapeDtypeStruct((q_n, k_n), jnp.float32),
        scratch_shapes=[pltpu.VMEM((qb, kp), jnp.float32)],
        compiler_params=pltpu.CompilerParams(
            dimension_semantics=("parallel",),
        ),
    )(w0, s1)
    return out


# speedup vs baseline: 1.0930x; 1.0930x over previous
import functools

import jax
import jax.numpy as jnp
import numpy as np
from jax.experimental import pallas as pl
from jax.experimental.pallas import tpu as pltpu

_K_SEL = 5
_EPS = float(np.finfo(np.float32).tiny)
_NB = 4096
_CB = 25600


def _wexp_kernel(q_ref, n_ref, w_ref, s1_ref, *, n_blocks, k_valid):
    j = pl.program_id(0)
    q = q_ref[...]
    n = n_ref[...]

    q2 = jnp.sum(q * q, axis=1, keepdims=True)
    n2 = jax.lax.dot_general(
        jnp.ones((8, n.shape[1]), jnp.float32), n * n,
        (((1,), (1,)), ((), ())),
        preferred_element_type=jnp.float32)[0:1, :]
    qn = jax.lax.dot_general(
        q, n, (((1,), (1,)), ((), ())),
        preferred_element_type=jnp.float32)
    d2 = jnp.maximum(q2 + n2 - 2.0 * qn, 0.0)
    w = jnp.exp(-jnp.sqrt(d2))
    w_ref[...] = w

    @pl.when(j == 0)
    def _():
        s1_ref[...] = jnp.zeros_like(s1_ref)

    last = n_blocks - 1
    valid = k_valid - last * _NB

    if valid < _NB:
        @pl.when(j < last)
        def _():
            s1_ref[:, 0:1] = s1_ref[:, 0:1] + jnp.sum(
                w, axis=1, keepdims=True)

        @pl.when(j == last)
        def _():
            w_ref[:, valid:] = jnp.zeros_like(w_ref[:, valid:])
            s1_ref[:, 0:1] = s1_ref[:, 0:1] + jnp.sum(
                w_ref[...], axis=1, keepdims=True)
    else:
        s1_ref[:, 0:1] = s1_ref[:, 0:1] + jnp.sum(w, axis=1, keepdims=True)


def _sweep_kernel(w0_ref, s1_ref, o_ref, *, kp, cb, qb):
    s = s1_ref[:, 0:1]
    n_chunks = kp // cb

    for it in range(_K_SEL):
        rinv = 1.0 / s
        if it == 0:
            def body0(c, acc, _rinv=rinv):
                sl = pl.ds(c * cb, cb)
                wv = w0_ref[:, sl]
                oh = wv * _rinv
                o_ref[:, sl] = oh
                wn = wv * jnp.maximum(1.0 - oh, _EPS)
                w0_ref[:, sl] = wn
                return acc + jnp.sum(wn, axis=1, keepdims=True)

            s = jax.lax.fori_loop(
                0, n_chunks, body0, jnp.zeros((qb, 1), jnp.float32))
        elif it < _K_SEL - 1:
            def body(c, acc, _rinv=rinv):
                sl = pl.ds(c * cb, cb)
                wv = w0_ref[:, sl]
                oh = wv * _rinv
                o_ref[:, sl] = o_ref[:, sl] + oh
                wn = wv * jnp.maximum(1.0 - oh, _EPS)
                w0_ref[:, sl] = wn
                return acc + jnp.sum(wn, axis=1, keepdims=True)

            s = jax.lax.fori_loop(
                0, n_chunks, body, jnp.zeros((qb, 1), jnp.float32))
        else:
            def body_last(c, acc, _rinv=rinv):
                sl = pl.ds(c * cb, cb)
                o_ref[:, sl] = o_ref[:, sl] + w0_ref[:, sl] * _rinv
                return acc

            jax.lax.fori_loop(0, n_chunks, body_last, 0)


def kernel(query, neighbors):
    q_n, d = query.shape
    k_n, _ = neighbors.shape

    n_blocks = -(-k_n // _NB)
    kp = n_blocks * _NB
    cb = _CB if kp % _CB == 0 else _NB
    qb = 32
    while q_n % qb:
        qb //= 2

    w0, s1 = pl.pallas_call(
        functools.partial(_wexp_kernel, n_blocks=n_blocks, k_valid=k_n),
        grid=(n_blocks,),
        in_specs=[
            pl.BlockSpec((q_n, d), lambda j: (0, 0)),
            pl.BlockSpec((_NB, d), lambda j: (j, 0)),
        ],
        out_specs=[
            pl.BlockSpec((q_n, _NB), lambda j: (0, j)),
            pl.BlockSpec((q_n, 128), lambda j: (0, 0)),
        ],
        out_shape=[
            jax.ShapeDtypeStruct((q_n, kp), jnp.float32),
            jax.ShapeDtypeStruct((q_n, 128), jnp.float32),
        ],
        compiler_params=pltpu.CompilerParams(
            dimension_semantics=("arbitrary",),
        ),
    )(query, neighbors)

    out = pl.pallas_call(
        functools.partial(_sweep_kernel, kp=kp, cb=cb, qb=qb),
        grid=(q_n // qb,),
        in_specs=[
            pl.BlockSpec((qb, kp), lambda i: (i, 0)),
            pl.BlockSpec((qb, 128), lambda i: (i, 0)),
        ],
        out_specs=pl.BlockSpec((qb, kp), lambda i: (i, 0)),
        out_shape=jax.ShapeDtypeStruct((q_n, k_n), jnp.float32),
        compiler_params=pltpu.CompilerParams(
            dimension_semantics=("parallel",),
        ),
    )(w0, s1)
    return out
